# Initial kernel scaffold; baseline (speedup 1.0000x reference)
import jax
import jax.numpy as jnp
from jax.experimental import pallas as pl


def kernel(logits):
    raise NotImplementedError("write your pallas kernel here")


# speedup vs baseline: 16.4758x; 16.4758x over previous
import jax
import jax.numpy as jnp
from jax import lax
from jax.experimental import pallas as pl

_B, _T, _V = 64, 8, 100000
_K = 256
_ROWS = _B * _T
_RB = 8


def _row_kernel(x_ref, g_ref, probs_ref, tok_ref):
    x = x_ref[...]
    i = lax.bitcast_convert_type(x, jnp.int32)
    m = lax.shift_right_arithmetic(i, 31)
    u = lax.bitcast_convert_type(i ^ (m | jnp.int32(-2147483648)), jnp.uint32)

    def body(k, ans):
        cand = ans | (jnp.uint32(0x80000000) >> k)
        cnt = jnp.sum((u >= cand).astype(jnp.int32), axis=1, keepdims=True)
        return jnp.where(cnt >= _K, cand, ans)

    ans = lax.fori_loop(0, 32, body, jnp.zeros((_RB, 1), jnp.uint32))

    neg = (ans & jnp.uint32(0x80000000)) != 0
    fbits = jnp.where(neg, ans ^ jnp.uint32(0x80000000), ~ans)
    thresh = lax.bitcast_convert_type(fbits, jnp.float32)

    mask = x >= thresh
    xm = jnp.where(mask, x, -jnp.inf)
    rowmax = jnp.max(xm, axis=1, keepdims=True)
    e = jnp.exp(xm - rowmax)
    s = jnp.sum(e, axis=1, keepdims=True)
    probs_ref[...] = e / s

    y = jnp.where(mask, g_ref[...] + x, -jnp.inf)
    ymax = jnp.max(y, axis=1, keepdims=True)
    lane = lax.broadcasted_iota(jnp.int32, (_RB, _V), 1)
    idx = jnp.where(y == ymax, lane, _V)
    tok_ref[...] = jnp.min(idx, axis=1, keepdims=True)


def kernel(logits):
    x = logits.reshape(_ROWS, _V)
    skey = jax.random.fold_in(jax.random.key(0), 1)
    g = jax.random.gumbel(skey, (_B, _T, _V), jnp.float32).reshape(_ROWS, _V)
    probs, toks = pl.pallas_call(
        _row_kernel,
        grid=(_ROWS // _RB,),
        in_specs=[
            pl.BlockSpec((_RB, _V), lambda j: (j, 0)),
            pl.BlockSpec((_RB, _V), lambda j: (j, 0)),
        ],
        out_specs=[
            pl.BlockSpec((_RB, _V), lambda j: (j, 0)),
            pl.BlockSpec((_RB, 1), lambda j: (j, 0)),
        ],
        out_shape=[
            jax.ShapeDtypeStruct((_ROWS, _V), jnp.float32),
            jax.ShapeDtypeStruct((_ROWS, 1), jnp.int32),
        ],
    )(x, g)
    return toks.reshape(_B, _T), probs.reshape(_B, _T, _V)


# speedup vs baseline: 17.4353x; 1.0582x over previous
import jax
import jax.numpy as jnp
from jax import lax
from jax.experimental import pallas as pl

_B, _T, _V = 64, 8, 100000
_K = 256
_ROWS = _B * _T
_RB = 8


def _sortable(x):
    i = lax.bitcast_convert_type(x, jnp.int32)
    m = lax.shift_right_arithmetic(i, 31)
    return lax.bitcast_convert_type(i ^ (m | jnp.int32(-2147483648)), jnp.uint32)


def _row_kernel(x_ref, g_ref, probs_ref, tok_ref):
    x = x_ref[...]
    u = _sortable(x)

    def top2_body(w, carry):
        m1, m2 = carry
        c = x_ref[:, pl.ds(pl.multiple_of(w * 128, 128), 128)]
        ge1 = c > m1
        m2 = jnp.maximum(m2, jnp.where(ge1, m1, c))
        m1 = jnp.where(ge1, c, m1)
        return m1, m2

    ninf = jnp.full((_RB, 128), -jnp.inf, jnp.float32)
    _, m2 = lax.fori_loop(0, _V // 128, top2_body, (ninf, ninf))
    t_lb = jnp.min(m2, axis=1, keepdims=True)
    rmax = jnp.max(x, axis=1, keepdims=True)

    lo0 = _sortable(t_lb)
    hi0 = _sortable(rmax)

    def cond(carry):
        lo, hi = carry
        return jnp.any(lo < hi)

    def body(carry):
        lo, hi = carry
        mid = lo + ((hi - lo + jnp.uint32(1)) >> 1)
        cnt = jnp.sum((u >= mid).astype(jnp.int32), axis=1, keepdims=True)
        big = cnt >= _K
        lo = jnp.where(big, mid, lo)
        hi = jnp.where(big, hi, mid - jnp.uint32(1))
        return lo, hi

    ans, _ = lax.while_loop(cond, body, (lo0, hi0))

    neg = (ans & jnp.uint32(0x80000000)) != 0
    fbits = jnp.where(neg, ans ^ jnp.uint32(0x80000000), ~ans)
    thresh = lax.bitcast_convert_type(fbits, jnp.float32)

    mask = x >= thresh
    xm = jnp.where(mask, x, -jnp.inf)
    rowmax = jnp.max(xm, axis=1, keepdims=True)
    e = jnp.exp(xm - rowmax)
    s = jnp.sum(e, axis=1, keepdims=True)
    probs_ref[...] = e / s

    y = jnp.where(mask, g_ref[...] + x, -jnp.inf)
    ymax = jnp.max(y, axis=1, keepdims=True)
    lane = lax.broadcasted_iota(jnp.int32, (_RB, _V), 1)
    idx = jnp.where(y == ymax, lane, _V)
    tok_ref[...] = jnp.min(idx, axis=1, keepdims=True)


def kernel(logits):
    x = logits.reshape(_ROWS, _V)
    skey = jax.random.fold_in(jax.random.key(0), 1)
    g = jax.random.gumbel(skey, (_B, _T, _V), jnp.float32).reshape(_ROWS, _V)
    probs, toks = pl.pallas_call(
        _row_kernel,
        grid=(_ROWS // _RB,),
        in_specs=[
            pl.BlockSpec((_RB, _V), lambda j: (j, 0)),
            pl.BlockSpec((_RB, _V), lambda j: (j, 0)),
        ],
        out_specs=[
            pl.BlockSpec((_RB, _V), lambda j: (j, 0)),
            pl.BlockSpec((_RB, 1), lambda j: (j, 0)),
        ],
        out_shape=[
            jax.ShapeDtypeStruct((_ROWS, _V), jnp.float32),
            jax.ShapeDtypeStruct((_ROWS, 1), jnp.int32),
        ],
    )(x, g)
    return toks.reshape(_B, _T), probs.reshape(_B, _T, _V)
